# baseline (speedup 1.0000x reference)
import functools

import jax
import jax.numpy as jnp
from jax import lax
from jax.experimental import pallas as pl
from jax.experimental.pallas import tpu as pltpu
from jax.experimental.pallas import tpu_sc as plsc

D_MODEL = 768
LANES = 16
VPR = D_MODEL // LANES
NUM_CORES = 2
NUM_SUBCORES = 16
NW = NUM_CORES * NUM_SUBCORES
CHUNK = 32
NBUF = 3


def _emb_body(n_batch, seq_len, tok_hbm, pos_hbm, idx_hbm, out_hbm,
              idx_v, pos_v, rows0, rows1, rows2,
              sg0, sg1, sg2, ss0, ss1, ss2, sp, si):
    ppw = seq_len // NW
    hpw = ppw // CHUNK
    n_chunks = n_batch * hpw

    cid = lax.axis_index("c")
    sid = lax.axis_index("s")
    wid = sid * NUM_CORES + cid
    t0 = wid * ppw

    posd = pltpu.async_copy(pos_hbm.at[pl.ds(t0, ppw)], pos_v, sp)
    idxd = [
        pltpu.async_copy(
            idx_hbm.at[pl.ds(b * NW * hpw + wid * hpw, hpw)],
            idx_v.at[pl.ds(b * hpw, hpw)], si)
        for b in range(n_batch)
    ]
    for d in idxd:
        d.wait()

    rows = [rows0, rows1, rows2]
    sg = [sg0, sg1, sg2]
    ss = [ss0, ss1, ss2]
    gd = [None] * NBUF
    sd = [None] * NBUF

    def start(k):
        b = k % NBUF
        gd[b] = pltpu.async_copy(tok_hbm.at[idx_v.at[k]], rows[b], sg[b])

    def run_add(rows_v, p0):
        @plsc.parallel_loop(0, CHUNK, 1, unroll=2)
        def add_body(r):
            rrow = rows_v.at[r]
            prow = pos_v.at[p0 + r]
            for c in range(VPR):
                plsc.addupdate(rrow.at[pl.ds(c * LANES, LANES)],
                               prow[pl.ds(c * LANES, LANES)])

    start(0)
    start(1)
    for k in range(n_chunks):
        b = k % NBUF
        with jax.named_scope(f"waitG{k}"):
            gd[b].wait()
            if k == 0:
                posd.wait()
        with jax.named_scope(f"waitS{k}"):
            if k + 2 < n_chunks:
                if k >= 1:
                    sd[(k + 2) % NBUF].wait()
                start(k + 2)
        with jax.named_scope(f"add{k}"):
            run_add(rows[b], (k % hpw) * CHUNK)
        out_row = (k // hpw) * seq_len + t0 + (k % hpw) * CHUNK
        sd[b] = pltpu.async_copy(
            rows[b], out_hbm.at[pl.ds(out_row, CHUNK)], ss[b])
    with jax.named_scope("drain"):
        for b in range(NBUF):
            sd[b].wait()


@jax.jit
def kernel(x, token_table, pos_table):
    B, T = x.shape
    n_rows = B * T
    ppw = T // NW
    n_chunks = n_rows // (NW * CHUNK)

    idx = x.astype(jnp.int32).reshape(n_rows // CHUNK, CHUNK)

    mesh = plsc.VectorSubcoreMesh(
        core_axis_name="c", subcore_axis_name="s")
    run = pl.kernel(
        functools.partial(_emb_body, B, T),
        out_type=jax.ShapeDtypeStruct((n_rows, D_MODEL), jnp.float32),
        mesh=mesh,
        scratch_types=[
            pltpu.VMEM((n_chunks, CHUNK), jnp.int32),
            pltpu.VMEM((ppw, D_MODEL), jnp.float32),
            pltpu.VMEM((CHUNK, D_MODEL), jnp.float32),
            pltpu.VMEM((CHUNK, D_MODEL), jnp.float32),
            pltpu.VMEM((CHUNK, D_MODEL), jnp.float32),
            pltpu.SemaphoreType.DMA,
            pltpu.SemaphoreType.DMA,
            pltpu.SemaphoreType.DMA,
            pltpu.SemaphoreType.DMA,
            pltpu.SemaphoreType.DMA,
            pltpu.SemaphoreType.DMA,
            pltpu.SemaphoreType.DMA,
            pltpu.SemaphoreType.DMA,
        ],
    )
    out = run(token_table, pos_table, idx)
    return out.reshape(B, T, D_MODEL)


# speedup vs baseline: 1.0518x; 1.0518x over previous
import functools

import jax
import jax.numpy as jnp
from jax import lax
from jax.experimental import pallas as pl
from jax.experimental.pallas import tpu as pltpu
from jax.experimental.pallas import tpu_sc as plsc

D_MODEL = 768
LANES = 16
VPR = D_MODEL // LANES
NUM_CORES = 2
NUM_SUBCORES = 16
NW = NUM_CORES * NUM_SUBCORES
CHUNK = 32
NBUF = 3


def _emb_body(n_batch, seq_len, tok_hbm, x_hbm, pos_hbm, out_hbm,
              idx_v, pos_v, rows0, rows1, rows2,
              sg0, sg1, sg2, ss0, ss1, ss2, sp, si):
    ppw = seq_len // NW
    hpw = ppw // CHUNK
    n_chunks = n_batch * hpw

    cid = lax.axis_index("c")
    sid = lax.axis_index("s")
    wid = sid * NUM_CORES + cid
    t0 = wid * ppw

    posd = pltpu.async_copy(pos_hbm.at[pl.ds(t0, ppw)], pos_v, sp)
    idxd = [
        pltpu.async_copy(x_hbm.at[b, pl.ds(t0, ppw)], idx_v.at[b], si)
        for b in range(n_batch)
    ]
    for d in idxd:
        d.wait()

    rows = [rows0, rows1, rows2]
    sg = [sg0, sg1, sg2]
    ss = [ss0, ss1, ss2]
    gd = [None] * NBUF
    sd = [None] * NBUF

    def start(k):
        b = k % NBUF
        gd[b] = pltpu.async_copy(
            tok_hbm.at[idx_v.at[k // hpw, pl.ds((k % hpw) * CHUNK, CHUNK)]],
            rows[b], sg[b])

    def run_add(rows_v, p0):
        @plsc.parallel_loop(0, CHUNK, 1, unroll=2)
        def add_body(r):
            rrow = rows_v.at[r]
            prow = pos_v.at[p0 + r]
            for c in range(VPR):
                plsc.addupdate(rrow.at[pl.ds(c * LANES, LANES)],
                               prow[pl.ds(c * LANES, LANES)])

    start(0)
    start(1)
    for k in range(n_chunks):
        b = k % NBUF
        gd[b].wait()
        if k == 0:
            posd.wait()
        run_add(rows[b], (k % hpw) * CHUNK)
        sd[b] = pltpu.async_copy(
            rows[b],
            out_hbm.at[k // hpw, pl.ds(t0 + (k % hpw) * CHUNK, CHUNK)],
            ss[b])
        if k + 2 < n_chunks:
            if k >= 1:
                sd[(k + 2) % NBUF].wait()
            start(k + 2)
    for b in range(NBUF):
        sd[b].wait()


@jax.jit
def kernel(x, token_table, pos_table):
    B, T = x.shape
    ppw = T // NW

    mesh = plsc.VectorSubcoreMesh(
        core_axis_name="c", subcore_axis_name="s")
    run = pl.kernel(
        functools.partial(_emb_body, B, T),
        out_type=jax.ShapeDtypeStruct((B, T, D_MODEL), jnp.float32),
        mesh=mesh,
        scratch_types=[
            pltpu.VMEM((B, ppw), jnp.int32),
            pltpu.VMEM((ppw, D_MODEL), jnp.float32),
            pltpu.VMEM((CHUNK, D_MODEL), jnp.float32),
            pltpu.VMEM((CHUNK, D_MODEL), jnp.float32),
            pltpu.VMEM((CHUNK, D_MODEL), jnp.float32),
            pltpu.SemaphoreType.DMA,
            pltpu.SemaphoreType.DMA,
            pltpu.SemaphoreType.DMA,
            pltpu.SemaphoreType.DMA,
            pltpu.SemaphoreType.DMA,
            pltpu.SemaphoreType.DMA,
            pltpu.SemaphoreType.DMA,
            pltpu.SemaphoreType.DMA,
        ],
    )
    return run(token_table, x.astype(jnp.int32), pos_table)


# speedup vs baseline: 1.0852x; 1.0318x over previous
import functools

import jax
import jax.numpy as jnp
from jax import lax
from jax.experimental import pallas as pl
from jax.experimental.pallas import tpu as pltpu
from jax.experimental.pallas import tpu_sc as plsc

D_MODEL = 768
LANES = 16
VPR = D_MODEL // LANES
NUM_CORES = 2
NUM_SUBCORES = 16
NW = NUM_CORES * NUM_SUBCORES
CHUNK = 32


def _emb_body(n_chunks, seq_len, tok_hbm, pos_hbm, idx_hbm, out_hbm,
              idx_v, rows0, rows1, pos0, pos1,
              sg0, sg1, sp0, sp1, ss0, ss1):
    cid = lax.axis_index("c")
    sid = lax.axis_index("s")
    wid = sid * NUM_CORES + cid
    base = wid * n_chunks
    pltpu.sync_copy(idx_hbm.at[pl.ds(base, n_chunks)], idx_v)
    t0 = (wid * n_chunks * CHUNK) % seq_len

    rows = [rows0, rows1]
    pos = [pos0, pos1]
    sg = [sg0, sg1]
    sp = [sp0, sp1]
    ss = [ss0, ss1]
    gd = [None, None]
    pd = [None, None]
    sd = [None, None]

    def make_add(rows_v, pos_v):
        def add_body(r, _):
            for c in range(VPR):
                plsc.addupdate(rows_v.at[r, pl.ds(c * LANES, LANES)],
                               pos_v[r, pl.ds(c * LANES, LANES)])
            return 0
        return add_body

    def start(k):
        b = k % 2
        gd[b] = pltpu.async_copy(tok_hbm.at[idx_v.at[k]], rows[b], sg[b])
        pd[b] = pltpu.async_copy(
            pos_hbm.at[pl.ds(t0 + k * CHUNK, CHUNK)], pos[b], sp[b])

    start(0)
    for k in range(n_chunks):
        b = k % 2
        gd[b].wait()
        pd[b].wait()
        if k + 1 < n_chunks:
            if k >= 1:
                sd[(k + 1) % 2].wait()
            start(k + 1)
        lax.fori_loop(0, CHUNK, make_add(rows[b], pos[b]), 0)
        sd[b] = pltpu.async_copy(
            rows[b], out_hbm.at[pl.ds((base + k) * CHUNK, CHUNK)], ss[b])
    sd[0].wait()
    sd[1].wait()


@jax.jit
def kernel(x, token_table, pos_table):
    B, T = x.shape
    n_rows = B * T
    n_chunks = n_rows // (NW * CHUNK)

    idx = x.astype(jnp.int32).reshape(NW * n_chunks, CHUNK)

    mesh = plsc.VectorSubcoreMesh(
        core_axis_name="c", subcore_axis_name="s")
    run = pl.kernel(
        functools.partial(_emb_body, n_chunks, T),
        out_type=jax.ShapeDtypeStruct((n_rows, D_MODEL), jnp.float32),
        mesh=mesh,
        scratch_types=[
            pltpu.VMEM((n_chunks, CHUNK), jnp.int32),
            pltpu.VMEM((CHUNK, D_MODEL), jnp.float32),
            pltpu.VMEM((CHUNK, D_MODEL), jnp.float32),
            pltpu.VMEM((CHUNK, D_MODEL), jnp.float32),
            pltpu.VMEM((CHUNK, D_MODEL), jnp.float32),
            pltpu.SemaphoreType.DMA,
            pltpu.SemaphoreType.DMA,
            pltpu.SemaphoreType.DMA,
            pltpu.SemaphoreType.DMA,
            pltpu.SemaphoreType.DMA,
            pltpu.SemaphoreType.DMA,
        ],
    )
    out = run(token_table, pos_table, idx)
    return out.reshape(B, T, D_MODEL)
